# baseline (speedup 1.0000x reference)
import jax
import jax.numpy as jnp
from jax import lax
from jax.experimental import pallas as pl
from jax.experimental.pallas import tpu as pltpu
from jax.experimental.pallas import tpu_sc as plsc

NC, NS, LANES = 2, 16, 16
NW = NC * NS
KL = 32
KA = 48
D = 256
DH = 128


def _prep_body(hid_ref, rel_ref, wsT, wrT, wqrT, bqr, a_ref, b_ref, c_ref):
    h = hid_ref[...]
    r = rel_ref[...]
    a_ref[...] = jnp.dot(h, wsT[...], preferred_element_type=jnp.float32) + bqr[...]
    b_ref[...] = jnp.dot(r, wrT[...], preferred_element_type=jnp.float32)
    c_ref[...] = jnp.dot(r, wqrT[...], preferred_element_type=jnp.float32)


def _final_body(ph0_ref, ph1_ref, whT1, whT2, out_ref):
    a = ph0_ref[0] + ph0_ref[1]
    b = ph1_ref[0] + ph1_ref[1]
    out_ref[...] = (jnp.dot(a, whT1[...], preferred_element_type=jnp.float32)
                    + jnp.dot(b, whT2[...], preferred_element_type=jnp.float32))


def _make_alpha_body(n_edge, ept):
    nchunk = ept // KL

    def body(suba, rela_i, ridxa, qrel_h, a_h, b_h, c_h, wal_h, bal_h,
             alpha_out,
             qrel_v, wal_v, bal_v, sub_all, rel_all, cidx_all, accbuf,
             ta0_v, tb0_v, tc0_v, ta1_v, tb1_v, tc1_v, alpha_v, sem0, sem1):
        cid = lax.axis_index("c")
        sid = lax.axis_index("s")
        w = cid * NS + sid
        sems = (sem0, sem1)
        tas = (ta0_v, ta1_v)
        tbs = (tb0_v, tb1_v)
        tcs = (tc0_v, tc1_v)

        tbase = pl.multiple_of(w * ept, 8)
        pltpu.sync_copy(qrel_h, qrel_v)
        pltpu.sync_copy(wal_h, wal_v)
        pltpu.sync_copy(bal_h, bal_v)
        pltpu.sync_copy(suba.at[pl.ds(tbase, ept)], sub_all)
        pltpu.sync_copy(rela_i.at[pl.ds(tbase, ept)], rel_all)
        pltpu.sync_copy(ridxa.at[pl.ds(tbase, ept)], cidx_all)
        iot = lax.iota(jnp.int32, 16)

        def mkcidx(t, carry):
            sl = pl.ds(t * 16, 16)
            cidx_all[sl] = plsc.load_gather(qrel_v, [cidx_all[sl]])
            return carry
        lax.fori_loop(0, ept // 16, mkcidx, 0)

        wal_regs = [wal_v[pl.ds(j * 16, 16)] for j in range(16)]

        def fire(c, b):
            g = lax.rem(c, nchunk)
            sl = pl.ds(g * KL, KL)
            pltpu.async_copy(a_h.at[sub_all.at[sl]], tas[b], sems[b])
            pltpu.async_copy(b_h.at[rel_all.at[sl]], tbs[b], sems[b])
            pltpu.async_copy(c_h.at[cidx_all.at[sl]], tcs[b], sems[b])

        def wait_rows(b):
            sl = pl.ds(0, KL)
            pltpu.make_async_copy(a_h.at[sub_all.at[sl]], tas[b], sems[b]).wait()
            pltpu.make_async_copy(b_h.at[rel_all.at[sl]], tbs[b], sems[b]).wait()
            pltpu.make_async_copy(c_h.at[cidx_all.at[sl]], tcs[b], sems[b]).wait()

        def compute(g, b):
            base = w * ept + g * KL
            ta, tb, tc = tas[b], tbs[b], tcs[b]

            def group_alpha(t, carry2):
                def edge_acc(e, carry3):
                    i = t * 16 + e
                    acc = jnp.zeros((16,), jnp.float32)
                    for jj in range(4):
                        for u in range(4):
                            j = jj * 4 + u
                            sl = pl.ds(j * 16, 16)
                            pre = ta[i, sl] + tb[i, sl] + tc[i, sl]
                            acc = acc + jnp.maximum(pre, 0.0) * wal_regs[j]
                    accbuf[e, :] = acc
                    return carry3
                lax.fori_loop(0, 16, edge_acc, 0)

                s = jnp.zeros((16,), jnp.float32)
                for j in range(16):
                    s = s + plsc.load_gather(
                        accbuf, [iot, jnp.full((16,), j, jnp.int32)])
                av = 1.0 / (1.0 + jnp.exp(-(s + bal_v[...])))
                eid = (base + t * 16) + iot
                av = jnp.where(eid < n_edge, av, 0.0)
                alpha_v[pl.ds(g * KL + t * 16, 16)] = av
                return carry2
            lax.fori_loop(0, KL // 16, group_alpha, 0)

        fire(0, 0)

        def pair(p, carry):
            fire(2 * p + 1, 1)
            wait_rows(0)
            compute(2 * p, 0)
            fire(2 * p + 2, 0)
            wait_rows(1)
            compute(2 * p + 1, 1)
            return carry
        lax.fori_loop(0, nchunk // 2, pair, 0)
        wait_rows(0)

        pltpu.sync_copy(alpha_v, alpha_out.at[pl.ds(tbase, ept)])

    return body


def _make_agg_body(ept, npad):
    nchunk = ept // KA
    rows_per_tile = npad // NS
    zrows = 32

    def body(ecols, alpha_h, hm1_h, rm1_h, hm2_h, rm2_h,
             ph0, ph1,
             agg, e0_v, e1_v, sub0_v, sub1_v, rel0_v, rel1_v, obj0_v, obj1_v,
             av0_v, av1_v, hm0_v, hm1_v_, rm0_v, rm1_v_, msg0_v, msg1_v, zbuf,
             semi0, semi1, semg0, semg1, sems0, sems1):
        cid = lax.axis_index("c")
        sid = lax.axis_index("s")
        w = cid * NS + sid
        row0 = sid * rows_per_tile
        semi = (semi0, semi1)
        semg = (semg0, semg1)
        sems = (sems0, sems1)
        e_v = (e0_v, e1_v)
        subs = (sub0_v, sub1_v)
        rels = (rel0_v, rel1_v)
        objs = (obj0_v, obj1_v)
        avs = (av0_v, av1_v)
        hms = (hm0_v, hm1_v_)
        rms = (rm0_v, rm1_v_)
        msgs = (msg0_v, msg1_v)
        tbase = pl.multiple_of(w * ept, 8)

        def zrow(r, carry):
            for j in range(8):
                zbuf[r, pl.ds(j * 16, 16)] = jnp.zeros((16,), jnp.float32)
            return carry
        lax.fori_loop(0, zrows, zrow, 0)

        def zero_agg():
            for q in range(rows_per_tile // zrows):
                pltpu.sync_copy(zbuf, agg.at[pl.ds(row0 + q * zrows, zrows)])

        def run_pass(hm_h, rm_h, pout):
            def fire_idx(c, b):
                g = lax.rem(c, nchunk)
                pltpu.async_copy(ecols.at[w, g], e_v[b], semi[b])
                pltpu.async_copy(alpha_h.at[pl.ds(tbase + g * KA, KA)],
                                 avs[b], semi[b])

            def wait_idx(b):
                pltpu.make_async_copy(ecols.at[w, 0], e_v[b], semi[b]).wait()
                pltpu.make_async_copy(alpha_h.at[pl.ds(0, KA)], avs[b],
                                      semi[b]).wait()

            def fire_rows(c, b):
                for t in range(KA // 16):
                    sl = pl.ds(t * 16, 16)
                    subs[b][sl] = e_v[b][0, sl]
                    rels[b][sl] = e_v[b][1, sl]
                pltpu.async_copy(hm_h.at[subs[b]], hms[b], semg[b])
                pltpu.async_copy(rm_h.at[rels[b]], rms[b], semg[b])

            def wait_rows(b):
                pltpu.make_async_copy(hm_h.at[subs[b]], hms[b], semg[b]).wait()
                pltpu.make_async_copy(rm_h.at[rels[b]], rms[b], semg[b]).wait()

            def wait_scat(b):
                pltpu.make_async_copy(msgs[b], agg.at[objs[b]], sems[b]).wait()

            def compute_scatter(g, b, first):
                if not first:
                    wait_scat(b)
                hm, rm, msg = hms[b], rms[b], msgs[b]

                def group_msg(t, carry2):
                    sl16 = pl.ds(t * 16, 16)
                    av = avs[b][sl16]
                    objs[b][sl16] = e_v[b][2, sl16]
                    for e in range(16):
                        i = t * 16 + e
                        a = av[e]
                        for j in range(8):
                            sl = pl.ds(j * 16, 16)
                            msg[i, sl] = a * (hm[i, sl] + rm[i, sl])
                    return carry2
                lax.fori_loop(0, KA // 16, group_msg, 0)
                pltpu.async_copy(msg, agg.at[objs[b]], sems[b], add=True)

            fire_idx(0, 0)
            fire_idx(1, 1)
            wait_idx(0)
            fire_rows(0, 0)
            wait_idx(1)
            fire_rows(1, 1)
            wait_rows(0)
            compute_scatter(0, 0, True)
            fire_idx(2, 0)
            wait_idx(0)
            fire_rows(2, 0)
            wait_rows(1)
            compute_scatter(1, 1, True)
            fire_idx(3, 1)

            def pair2(p, carry):
                c0 = 2 * p + 2
                wait_idx(1)
                fire_rows(c0 + 1, 1)
                wait_rows(0)
                compute_scatter(c0, 0, False)
                fire_idx(c0 + 2, 0)
                wait_idx(0)
                fire_rows(c0 + 2, 0)
                wait_rows(1)
                compute_scatter(c0 + 1, 1, False)
                fire_idx(c0 + 3, 1)
                return carry
            lax.fori_loop(0, nchunk // 2 - 1, pair2, 0)
            wait_rows(0)
            wait_idx(1)
            wait_scat(0)
            wait_scat(1)
            plsc.subcore_barrier()
            pltpu.sync_copy(agg.at[pl.ds(row0, rows_per_tile)],
                            pout.at[cid, pl.ds(row0, rows_per_tile)])

        zero_agg()
        plsc.subcore_barrier()
        run_pass(hm1_h, rm1_h, ph0)
        zero_agg()
        plsc.subcore_barrier()
        run_pass(hm2_h, rm2_h, ph1)

    return body


def kernel(q_sub, q_rel, hidden, edges, nodes, old_nodes_new_idx, batchsize,
           rela_embed, Ws, Wr, Wqr, b_qr, w_alpha, b_alpha, W_h):
    n_node = nodes.shape[0]
    n_edge = edges.shape[0]
    n_rel = rela_embed.shape[0]
    f32 = jnp.float32

    sub = edges[:, 4].astype(jnp.int32)
    rel = edges[:, 2].astype(jnp.int32)
    obj = edges[:, 5].astype(jnp.int32)
    ridx = edges[:, 0].astype(jnp.int32)

    npad = ((max(n_node, n_rel) + 255) // 256) * 256
    lcm = 2 * KL * KA // 16
    ept = ((n_edge + NW * lcm - 1) // (NW * lcm)) * lcm
    nchunk_a = ept // KA
    epad = ept * NW
    pad = epad - n_edge

    suba = jnp.pad(sub, (0, pad))
    rela_i = jnp.pad(rel, (0, pad))
    obja = jnp.pad(obj, (0, pad))
    ridxa = jnp.pad(ridx, (0, pad))

    def colpack(x):
        return x.reshape(NW, nchunk_a, KA)
    ecols = jnp.stack([colpack(suba), colpack(rela_i), colpack(obja)], axis=2)

    hid_p = jnp.pad(hidden.astype(f32), ((0, npad - n_node), (0, 0)))
    rel_p = jnp.pad(rela_embed.astype(f32), ((0, npad - n_rel), (0, 0)))

    nblk = npad // 256
    tbl_a, tbl_b, tbl_c = pl.pallas_call(
        _prep_body,
        grid=(nblk,),
        in_specs=[
            pl.BlockSpec((256, D), lambda i: (i, 0)),
            pl.BlockSpec((256, D), lambda i: (i, 0)),
            pl.BlockSpec((D, D), lambda i: (0, 0)),
            pl.BlockSpec((D, D), lambda i: (0, 0)),
            pl.BlockSpec((D, D), lambda i: (0, 0)),
            pl.BlockSpec((1, D), lambda i: (0, 0)),
        ],
        out_specs=[
            pl.BlockSpec((256, D), lambda i: (i, 0)),
            pl.BlockSpec((256, D), lambda i: (i, 0)),
            pl.BlockSpec((256, D), lambda i: (i, 0)),
        ],
        out_shape=[jax.ShapeDtypeStruct((npad, D), f32)] * 3,
    )(hid_p, rel_p, Ws.T.astype(f32), Wr.T.astype(f32), Wqr.T.astype(f32),
      b_qr.reshape(1, D).astype(f32))

    wal = w_alpha.reshape(-1).astype(f32)
    bal = jnp.broadcast_to(b_alpha.astype(f32), (16,))

    mesh = plsc.VectorSubcoreMesh(core_axis_name="c", subcore_axis_name="s",
                                  num_cores=NC, num_subcores=NS)
    sc_params = pltpu.CompilerParams(needs_layout_passes=False)

    i32 = jnp.int32
    alpha_fn = pl.kernel(
        _make_alpha_body(n_edge, ept),
        out_type=jax.ShapeDtypeStruct((epad,), f32),
        mesh=mesh,
        compiler_params=sc_params,
        scratch_types=[
            pltpu.VMEM((q_rel.shape[0],), i32),
            pltpu.VMEM((D,), f32),
            pltpu.VMEM((16,), f32),
            pltpu.VMEM((ept,), i32),
            pltpu.VMEM((ept,), i32),
            pltpu.VMEM((ept,), i32),
            pltpu.VMEM((16, 16), f32),
            pltpu.VMEM((KL, D), f32),
            pltpu.VMEM((KL, D), f32),
            pltpu.VMEM((KL, D), f32),
            pltpu.VMEM((KL, D), f32),
            pltpu.VMEM((KL, D), f32),
            pltpu.VMEM((KL, D), f32),
            pltpu.VMEM((ept,), f32),
            pltpu.SemaphoreType.DMA,
            pltpu.SemaphoreType.DMA,
        ],
    )
    alphas = alpha_fn(suba, rela_i, ridxa, q_rel.astype(i32),
                      tbl_a, tbl_b, tbl_c, wal, bal)

    hm1 = hidden[:, :DH].astype(f32)
    hm2 = hidden[:, DH:].astype(f32)
    rm1 = rela_embed[:, :DH].astype(f32)
    rm2 = rela_embed[:, DH:].astype(f32)

    agg_fn = pl.kernel(
        _make_agg_body(ept, npad),
        out_type=(jax.ShapeDtypeStruct((NC, npad, DH), f32),
                  jax.ShapeDtypeStruct((NC, npad, DH), f32)),
        mesh=mesh,
        compiler_params=sc_params,
        scratch_types=[
            pltpu.VMEM_SHARED((npad, DH), f32),
            pltpu.VMEM((3, KA), i32),
            pltpu.VMEM((3, KA), i32),
            pltpu.VMEM((KA,), i32),
            pltpu.VMEM((KA,), i32),
            pltpu.VMEM((KA,), i32),
            pltpu.VMEM((KA,), i32),
            pltpu.VMEM((KA,), i32),
            pltpu.VMEM((KA,), i32),
            pltpu.VMEM((KA,), f32),
            pltpu.VMEM((KA,), f32),
            pltpu.VMEM((KA, DH), f32),
            pltpu.VMEM((KA, DH), f32),
            pltpu.VMEM((KA, DH), f32),
            pltpu.VMEM((KA, DH), f32),
            pltpu.VMEM((KA, DH), f32),
            pltpu.VMEM((KA, DH), f32),
            pltpu.VMEM((32, DH), f32),
            pltpu.SemaphoreType.DMA,
            pltpu.SemaphoreType.DMA,
            pltpu.SemaphoreType.DMA,
            pltpu.SemaphoreType.DMA,
            pltpu.SemaphoreType.DMA,
            pltpu.SemaphoreType.DMA,
        ],
    )
    ph0, ph1 = agg_fn(ecols, alphas, hm1, rm1, hm2, rm2)

    whT = W_h.T.astype(f32)
    out = pl.pallas_call(
        _final_body,
        grid=(nblk,),
        in_specs=[
            pl.BlockSpec((NC, 256, DH), lambda i: (0, i, 0)),
            pl.BlockSpec((NC, 256, DH), lambda i: (0, i, 0)),
            pl.BlockSpec((DH, D), lambda i: (0, 0)),
            pl.BlockSpec((DH, D), lambda i: (0, 0)),
        ],
        out_specs=pl.BlockSpec((256, D), lambda i: (i, 0)),
        out_shape=jax.ShapeDtypeStruct((npad, D), f32),
    )(ph0, ph1, whT[:DH], whT[DH:])

    return out[:n_node]


# speedup vs baseline: 1.0343x; 1.0343x over previous
import jax
import jax.numpy as jnp
from jax import lax
from jax.experimental import pallas as pl
from jax.experimental.pallas import tpu as pltpu
from jax.experimental.pallas import tpu_sc as plsc

NC, NS, LANES = 2, 16, 16
NW = NC * NS
K = 48
D = 256
DH = 128


def _prep_body(hid_ref, rel_ref, wsT, wrT, wqrT, bqr, a_ref, b_ref, c_ref):
    h = hid_ref[...]
    r = rel_ref[...]
    a_ref[...] = jnp.dot(h, wsT[...], preferred_element_type=jnp.float32) + bqr[...]
    b_ref[...] = jnp.dot(r, wrT[...], preferred_element_type=jnp.float32)
    c_ref[...] = jnp.dot(r, wqrT[...], preferred_element_type=jnp.float32)


def _final_body(ph0_ref, ph1_ref, whT1, whT2, out_ref):
    a = ph0_ref[0] + ph0_ref[1]
    b = ph1_ref[0] + ph1_ref[1]
    out_ref[...] = (jnp.dot(a, whT1[...], preferred_element_type=jnp.float32)
                    + jnp.dot(b, whT2[...], preferred_element_type=jnp.float32))


def _make_alpha_body(n_edge, ept):
    nchunk = ept // K

    def body(ecols, qrel_h, a_h, b_h, c_h, wal_h, bal_h,
             alpha_out,
             qrel_v, wal_v, bal_v, e0_v, e1_v, sub0_v, sub1_v, rel0_v, rel1_v,
             cidx0_v, cidx1_v, accbuf,
             ta0_v, tb0_v, tc0_v, ta1_v, tb1_v, tc1_v, alpha_v, sem0, sem1):
        cid = lax.axis_index("c")
        sid = lax.axis_index("s")
        w = cid * NS + sid
        sems = (sem0, sem1)
        e_v = (e0_v, e1_v)
        subs = (sub0_v, sub1_v)
        rels = (rel0_v, rel1_v)
        cidxs = (cidx0_v, cidx1_v)
        tas = (ta0_v, ta1_v)
        tbs = (tb0_v, tb1_v)
        tcs = (tc0_v, tc1_v)

        pltpu.sync_copy(qrel_h, qrel_v)
        pltpu.sync_copy(wal_h, wal_v)
        pltpu.sync_copy(bal_h, bal_v)
        iot = lax.iota(jnp.int32, 16)

        def pf(c, b):
            g = lax.rem(c, nchunk)
            pltpu.sync_copy(ecols.at[w, g], e_v[b])
            for t in range(K // 16):
                sl = pl.ds(t * 16, 16)
                subs[b][sl] = e_v[b][0, sl]
                rels[b][sl] = e_v[b][1, sl]
                cidxs[b][sl] = plsc.load_gather(qrel_v, [e_v[b][2, sl]])
            pltpu.async_copy(a_h.at[subs[b]], tas[b], sems[b])
            pltpu.async_copy(b_h.at[rels[b]], tbs[b], sems[b])
            pltpu.async_copy(c_h.at[cidxs[b]], tcs[b], sems[b])

        def wait_rows(b):
            pltpu.make_async_copy(a_h.at[subs[b]], tas[b], sems[b]).wait()
            pltpu.make_async_copy(b_h.at[rels[b]], tbs[b], sems[b]).wait()
            pltpu.make_async_copy(c_h.at[cidxs[b]], tcs[b], sems[b]).wait()

        def compute(g, b):
            base = pl.multiple_of(w * ept + g * K, 8)
            ta, tb, tc = tas[b], tbs[b], tcs[b]

            def group_alpha(t, carry2):
                def edge_acc(e, carry3):
                    i = t * 16 + e

                    def feat(jj, acc):
                        for u in range(4):
                            sl = pl.ds(jj * 64 + u * 16, 16)
                            pre = ta[i, sl] + tb[i, sl] + tc[i, sl]
                            acc = acc + jnp.maximum(pre, 0.0) * wal_v[sl]
                        return acc
                    acc = lax.fori_loop(0, 4, feat, jnp.zeros((16,), jnp.float32))
                    accbuf[e, :] = acc
                    return carry3
                lax.fori_loop(0, 16, edge_acc, 0)

                s = jnp.zeros((16,), jnp.float32)
                for j in range(16):
                    s = s + plsc.load_gather(
                        accbuf, [iot, jnp.full((16,), j, jnp.int32)])
                av = 1.0 / (1.0 + jnp.exp(-(s + bal_v[...])))
                eid = (base + t * 16) + iot
                av = jnp.where(eid < n_edge, av, 0.0)
                alpha_v[pl.ds(g * K + t * 16, 16)] = av
                return carry2
            lax.fori_loop(0, K // 16, group_alpha, 0)

        pf(0, 0)

        def pair(p, carry):
            pf(2 * p + 1, 1)
            wait_rows(0)
            compute(2 * p, 0)
            pf(2 * p + 2, 0)
            wait_rows(1)
            compute(2 * p + 1, 1)
            return carry
        lax.fori_loop(0, nchunk // 2, pair, 0)
        wait_rows(0)

        pltpu.sync_copy(alpha_v, alpha_out.at[pl.ds(w * ept, ept)])

    return body


def _make_agg_body(ept, npad):
    nchunk = ept // K
    rows_per_tile = npad // NS
    zrows = 64

    def body(ecols, alpha_h, hm1_h, rm1_h, hm2_h, rm2_h,
             ph0, ph1,
             agg, e0_v, e1_v, sub0_v, sub1_v, rel0_v, rel1_v, obj0_v, obj1_v,
             av0_v, av1_v, hm0_v, hm1_v_, rm0_v, rm1_v_, msg0_v, msg1_v, zbuf,
             semg0, semg1, sems0, sems1):
        cid = lax.axis_index("c")
        sid = lax.axis_index("s")
        w = cid * NS + sid
        row0 = sid * rows_per_tile
        semg = (semg0, semg1)
        sems = (sems0, sems1)
        e_v = (e0_v, e1_v)
        subs = (sub0_v, sub1_v)
        rels = (rel0_v, rel1_v)
        objs = (obj0_v, obj1_v)
        avs = (av0_v, av1_v)
        hms = (hm0_v, hm1_v_)
        rms = (rm0_v, rm1_v_)
        msgs = (msg0_v, msg1_v)

        def zrow(r, carry):
            for j in range(8):
                zbuf[r, pl.ds(j * 16, 16)] = jnp.zeros((16,), jnp.float32)
            return carry
        lax.fori_loop(0, zrows, zrow, 0)

        def zero_agg():
            for q in range(rows_per_tile // zrows):
                pltpu.sync_copy(zbuf, agg.at[pl.ds(row0 + q * zrows, zrows)])

        def run_pass(hm_h, rm_h, pout):
            def pf(c, b):
                g = lax.rem(c, nchunk)
                base = pl.multiple_of(w * ept + g * K, 8)
                pltpu.sync_copy(ecols.at[w, g], e_v[b])
                pltpu.sync_copy(alpha_h.at[pl.ds(base, K)], avs[b])
                for t in range(K // 16):
                    sl = pl.ds(t * 16, 16)
                    subs[b][sl] = e_v[b][0, sl]
                    rels[b][sl] = e_v[b][1, sl]
                pltpu.async_copy(hm_h.at[subs[b]], hms[b], semg[b])
                pltpu.async_copy(rm_h.at[rels[b]], rms[b], semg[b])

            def wait_rows(b):
                pltpu.make_async_copy(hm_h.at[subs[b]], hms[b], semg[b]).wait()
                pltpu.make_async_copy(rm_h.at[rels[b]], rms[b], semg[b]).wait()

            def wait_scat(b):
                pltpu.make_async_copy(msgs[b], agg.at[objs[b]], sems[b]).wait()

            def compute_scatter(g, b, first):
                if not first:
                    wait_scat(b)
                hm, rm, msg = hms[b], rms[b], msgs[b]

                def edge_msg(i, carry2):
                    a = plsc.load_gather(avs[b], [jnp.full((16,), i, jnp.int32)])

                    def feat(jj, carry3):
                        for u in range(4):
                            sl = pl.ds(jj * 64 + u * 16, 16)
                            msg[i, sl] = a * (hm[i, sl] + rm[i, sl])
                        return carry3
                    lax.fori_loop(0, 2, feat, 0)
                    return carry2
                lax.fori_loop(0, K, edge_msg, 0)
                for t in range(K // 16):
                    sl = pl.ds(t * 16, 16)
                    objs[b][sl] = e_v[b][3, sl]
                pltpu.async_copy(msg, agg.at[objs[b]], sems[b], add=True)

            pf(0, 0)
            pf(1, 1)
            wait_rows(0)
            compute_scatter(0, 0, True)
            pf(2, 0)
            wait_rows(1)
            compute_scatter(1, 1, True)

            def pair2(p, carry):
                pf(2 * p + 3, 1)
                wait_rows(0)
                compute_scatter(2 * p + 2, 0, False)
                pf(2 * p + 4, 0)
                wait_rows(1)
                compute_scatter(2 * p + 3, 1, False)
                return carry
            lax.fori_loop(0, nchunk // 2 - 1, pair2, 0)
            wait_rows(0)
            wait_scat(0)
            wait_scat(1)
            plsc.subcore_barrier()
            pltpu.sync_copy(agg.at[pl.ds(row0, rows_per_tile)],
                            pout.at[cid, pl.ds(row0, rows_per_tile)])

        zero_agg()
        plsc.subcore_barrier()
        run_pass(hm1_h, rm1_h, ph0)
        zero_agg()
        plsc.subcore_barrier()
        run_pass(hm2_h, rm2_h, ph1)

    return body


def kernel(q_sub, q_rel, hidden, edges, nodes, old_nodes_new_idx, batchsize,
           rela_embed, Ws, Wr, Wqr, b_qr, w_alpha, b_alpha, W_h):
    n_node = nodes.shape[0]
    n_edge = edges.shape[0]
    n_rel = rela_embed.shape[0]
    f32 = jnp.float32

    sub = edges[:, 4].astype(jnp.int32)
    rel = edges[:, 2].astype(jnp.int32)
    obj = edges[:, 5].astype(jnp.int32)
    ridx = edges[:, 0].astype(jnp.int32)

    npad = ((max(n_node, n_rel) + 255) // 256) * 256
    ept = ((n_edge + NW * 2 * K - 1) // (NW * 2 * K)) * 2 * K
    nchunk = ept // K
    epad = ept * NW
    pad = epad - n_edge

    def colpack(x):
        return jnp.pad(x, (0, pad)).reshape(NW, nchunk, K)
    ecols = jnp.stack(
        [colpack(sub), colpack(rel), colpack(ridx), colpack(obj)], axis=2)

    hid_p = jnp.pad(hidden.astype(f32), ((0, npad - n_node), (0, 0)))
    rel_p = jnp.pad(rela_embed.astype(f32), ((0, npad - n_rel), (0, 0)))

    nblk = npad // 256
    tbl_a, tbl_b, tbl_c = pl.pallas_call(
        _prep_body,
        grid=(nblk,),
        in_specs=[
            pl.BlockSpec((256, D), lambda i: (i, 0)),
            pl.BlockSpec((256, D), lambda i: (i, 0)),
            pl.BlockSpec((D, D), lambda i: (0, 0)),
            pl.BlockSpec((D, D), lambda i: (0, 0)),
            pl.BlockSpec((D, D), lambda i: (0, 0)),
            pl.BlockSpec((1, D), lambda i: (0, 0)),
        ],
        out_specs=[
            pl.BlockSpec((256, D), lambda i: (i, 0)),
            pl.BlockSpec((256, D), lambda i: (i, 0)),
            pl.BlockSpec((256, D), lambda i: (i, 0)),
        ],
        out_shape=[jax.ShapeDtypeStruct((npad, D), f32)] * 3,
    )(hid_p, rel_p, Ws.T.astype(f32), Wr.T.astype(f32), Wqr.T.astype(f32),
      b_qr.reshape(1, D).astype(f32))

    wal = w_alpha.reshape(-1).astype(f32)
    bal = jnp.broadcast_to(b_alpha.astype(f32), (16,))

    mesh = plsc.VectorSubcoreMesh(core_axis_name="c", subcore_axis_name="s",
                                  num_cores=NC, num_subcores=NS)
    sc_params = pltpu.CompilerParams(needs_layout_passes=False)

    i32 = jnp.int32
    alpha_fn = pl.kernel(
        _make_alpha_body(n_edge, ept),
        out_type=jax.ShapeDtypeStruct((epad,), f32),
        mesh=mesh,
        compiler_params=sc_params,
        scratch_types=[
            pltpu.VMEM((q_rel.shape[0],), i32),
            pltpu.VMEM((D,), f32),
            pltpu.VMEM((16,), f32),
            pltpu.VMEM((4, K), i32),
            pltpu.VMEM((4, K), i32),
            pltpu.VMEM((K,), i32),
            pltpu.VMEM((K,), i32),
            pltpu.VMEM((K,), i32),
            pltpu.VMEM((K,), i32),
            pltpu.VMEM((K,), i32),
            pltpu.VMEM((K,), i32),
            pltpu.VMEM((16, 16), f32),
            pltpu.VMEM((K, D), f32),
            pltpu.VMEM((K, D), f32),
            pltpu.VMEM((K, D), f32),
            pltpu.VMEM((K, D), f32),
            pltpu.VMEM((K, D), f32),
            pltpu.VMEM((K, D), f32),
            pltpu.VMEM((ept,), f32),
            pltpu.SemaphoreType.DMA,
            pltpu.SemaphoreType.DMA,
        ],
    )
    alphas = alpha_fn(ecols, q_rel.astype(i32), tbl_a, tbl_b, tbl_c, wal, bal)

    hm1 = hidden[:, :DH].astype(f32)
    hm2 = hidden[:, DH:].astype(f32)
    rm1 = rela_embed[:, :DH].astype(f32)
    rm2 = rela_embed[:, DH:].astype(f32)

    agg_fn = pl.kernel(
        _make_agg_body(ept, npad),
        out_type=(jax.ShapeDtypeStruct((NC, npad, DH), f32),
                  jax.ShapeDtypeStruct((NC, npad, DH), f32)),
        mesh=mesh,
        compiler_params=sc_params,
        scratch_types=[
            pltpu.VMEM_SHARED((npad, DH), f32),
            pltpu.VMEM((4, K), i32),
            pltpu.VMEM((4, K), i32),
            pltpu.VMEM((K,), i32),
            pltpu.VMEM((K,), i32),
            pltpu.VMEM((K,), i32),
            pltpu.VMEM((K,), i32),
            pltpu.VMEM((K,), i32),
            pltpu.VMEM((K,), i32),
            pltpu.VMEM((K,), f32),
            pltpu.VMEM((K,), f32),
            pltpu.VMEM((K, DH), f32),
            pltpu.VMEM((K, DH), f32),
            pltpu.VMEM((K, DH), f32),
            pltpu.VMEM((K, DH), f32),
            pltpu.VMEM((K, DH), f32),
            pltpu.VMEM((K, DH), f32),
            pltpu.VMEM((64, DH), f32),
            pltpu.SemaphoreType.DMA,
            pltpu.SemaphoreType.DMA,
            pltpu.SemaphoreType.DMA,
            pltpu.SemaphoreType.DMA,
        ],
    )
    ph0, ph1 = agg_fn(ecols, alphas, hm1, rm1, hm2, rm2)

    whT = W_h.T.astype(f32)
    out = pl.pallas_call(
        _final_body,
        grid=(nblk,),
        in_specs=[
            pl.BlockSpec((NC, 256, DH), lambda i: (0, i, 0)),
            pl.BlockSpec((NC, 256, DH), lambda i: (0, i, 0)),
            pl.BlockSpec((DH, D), lambda i: (0, 0)),
            pl.BlockSpec((DH, D), lambda i: (0, 0)),
        ],
        out_specs=pl.BlockSpec((256, D), lambda i: (i, 0)),
        out_shape=jax.ShapeDtypeStruct((npad, D), f32),
    )(ph0, ph1, whT[:DH], whT[DH:])

    return out[:n_node]


# speedup vs baseline: 1.1666x; 1.1280x over previous
import jax
import jax.numpy as jnp
from jax import lax
from jax.experimental import pallas as pl
from jax.experimental.pallas import tpu as pltpu
from jax.experimental.pallas import tpu_sc as plsc

NC, NS, LANES = 2, 16, 16
NW = NC * NS
K = 48
D = 256
DH = 128


def _prep_body(hid_ref, rel_ref, wsT, wrT, wqrT, bqr, a_ref, b_ref, c_ref):
    h = hid_ref[...]
    r = rel_ref[...]
    a_ref[...] = jnp.dot(h, wsT[...], preferred_element_type=jnp.float32) + bqr[...]
    b_ref[...] = jnp.dot(r, wrT[...], preferred_element_type=jnp.float32)
    c_ref[...] = jnp.dot(r, wqrT[...], preferred_element_type=jnp.float32)


def _final_body(ph0_ref, ph1_ref, whT1, whT2, out_ref):
    a = ph0_ref[0] + ph0_ref[1]
    b = ph1_ref[0] + ph1_ref[1]
    out_ref[...] = (jnp.dot(a, whT1[...], preferred_element_type=jnp.float32)
                    + jnp.dot(b, whT2[...], preferred_element_type=jnp.float32))


def _make_alpha_body(n_edge, ept):
    nchunk = ept // K

    def body(ecols, qrel_h, a_h, b_h, c_h, wal_h, bal_h,
             alpha_out,
             qrel_v, wal_v, bal_v, e0_v, e1_v, sub0_v, sub1_v, rel0_v, rel1_v,
             cidx0_v, cidx1_v, accbuf,
             ta0_v, tb0_v, tc0_v, ta1_v, tb1_v, tc1_v, alpha_v, sem0, sem1):
        cid = lax.axis_index("c")
        sid = lax.axis_index("s")
        w = cid * NS + sid
        sems = (sem0, sem1)
        e_v = (e0_v, e1_v)
        subs = (sub0_v, sub1_v)
        rels = (rel0_v, rel1_v)
        cidxs = (cidx0_v, cidx1_v)
        tas = (ta0_v, ta1_v)
        tbs = (tb0_v, tb1_v)
        tcs = (tc0_v, tc1_v)

        pltpu.sync_copy(qrel_h, qrel_v)
        pltpu.sync_copy(wal_h, wal_v)
        pltpu.sync_copy(bal_h, bal_v)
        iot = lax.iota(jnp.int32, 16)

        def pf(c, b):
            g = lax.rem(c, nchunk)
            pltpu.sync_copy(ecols.at[w, g], e_v[b])
            for t in range(K // 16):
                sl = pl.ds(t * 16, 16)
                subs[b][sl] = e_v[b][0, sl]
                rels[b][sl] = e_v[b][1, sl]
                cidxs[b][sl] = plsc.load_gather(qrel_v, [e_v[b][2, sl]])
            pltpu.async_copy(a_h.at[subs[b]], tas[b], sems[b])
            pltpu.async_copy(b_h.at[rels[b]], tbs[b], sems[b])
            pltpu.async_copy(c_h.at[cidxs[b]], tcs[b], sems[b])

        def wait_rows(b):
            pltpu.make_async_copy(a_h.at[subs[b]], tas[b], sems[b]).wait()
            pltpu.make_async_copy(b_h.at[rels[b]], tbs[b], sems[b]).wait()
            pltpu.make_async_copy(c_h.at[cidxs[b]], tcs[b], sems[b]).wait()

        def compute(g, b):
            base = pl.multiple_of(w * ept + g * K, 8)
            ta, tb, tc = tas[b], tbs[b], tcs[b]

            def group_alpha(t, carry2):
                def edge_acc(e, carry3):
                    i = t * 16 + e

                    def feat(jj, acc):
                        for u in range(4):
                            sl = pl.ds(jj * 64 + u * 16, 16)
                            pre = ta[i, sl] + tb[i, sl] + tc[i, sl]
                            acc = acc + jnp.maximum(pre, 0.0) * wal_v[sl]
                        return acc
                    acc = lax.fori_loop(0, 4, feat, jnp.zeros((16,), jnp.float32))
                    accbuf[e, :] = acc
                    return carry3
                lax.fori_loop(0, 16, edge_acc, 0)

                s = jnp.zeros((16,), jnp.float32)
                for j in range(16):
                    s = s + plsc.load_gather(
                        accbuf, [iot, jnp.full((16,), j, jnp.int32)])
                av = 1.0 / (1.0 + jnp.exp(-(s + bal_v[...])))
                eid = (base + t * 16) + iot
                av = jnp.where(eid < n_edge, av, 0.0)
                alpha_v[pl.ds(g * K + t * 16, 16)] = av
                return carry2
            lax.fori_loop(0, K // 16, group_alpha, 0)

        pf(0, 0)

        def pair(p, carry):
            pf(2 * p + 1, 1)
            wait_rows(0)
            compute(2 * p, 0)
            pf(2 * p + 2, 0)
            wait_rows(1)
            compute(2 * p + 1, 1)
            return carry
        lax.fori_loop(0, nchunk // 2, pair, 0)
        wait_rows(0)

        pltpu.sync_copy(alpha_v, alpha_out.at[pl.ds(w * ept, ept)])

    return body


def _make_agg_body(ept, npad):
    nchunk = ept // K
    rows_per_tile = npad // NS
    zrows = 64

    def body(ecols, hm1_h, rm1_h, hm2_h, rm2_h,
             ph0, ph1,
             agg, e0_v, e1_v, sub0_v, sub1_v, rel0_v, rel1_v, obj0_v, obj1_v,
             hm0_v, hm1_v_, rm0_v, rm1_v_, msg0_v, msg1_v, zbuf,
             semg0, semg1, sems0, sems1):
        cid = lax.axis_index("c")
        sid = lax.axis_index("s")
        w = cid * NS + sid
        row0 = sid * rows_per_tile
        semg = (semg0, semg1)
        sems = (sems0, sems1)
        e_v = (e0_v, e1_v)
        subs = (sub0_v, sub1_v)
        rels = (rel0_v, rel1_v)
        objs = (obj0_v, obj1_v)
        hms = (hm0_v, hm1_v_)
        rms = (rm0_v, rm1_v_)
        msgs = (msg0_v, msg1_v)

        def zrow(r, carry):
            for j in range(8):
                zbuf[r, pl.ds(j * 16, 16)] = jnp.zeros((16,), jnp.float32)
            return carry
        lax.fori_loop(0, zrows, zrow, 0)

        def zero_agg():
            for q in range(rows_per_tile // zrows):
                pltpu.sync_copy(zbuf, agg.at[pl.ds(row0 + q * zrows, zrows)])

        def run_pass(hm_h, rm_h, pout):
            def pf(c, b):
                g = lax.rem(c, nchunk)
                pltpu.sync_copy(ecols.at[w, g], e_v[b])
                for t in range(K // 16):
                    sl = pl.ds(t * 16, 16)
                    subs[b][sl] = e_v[b][0, sl]
                    rels[b][sl] = e_v[b][1, sl]
                pltpu.async_copy(hm_h.at[subs[b]], hms[b], semg[b])
                pltpu.async_copy(rm_h.at[rels[b]], rms[b], semg[b])

            def wait_rows(b):
                pltpu.make_async_copy(hm_h.at[subs[b]], hms[b], semg[b]).wait()
                pltpu.make_async_copy(rm_h.at[rels[b]], rms[b], semg[b]).wait()

            def wait_scat(b):
                pltpu.make_async_copy(msgs[b], agg.at[objs[b]], sems[b]).wait()

            def compute_scatter(g, b, first):
                if not first:
                    wait_scat(b)
                hm, rm, msg = hms[b], rms[b], msgs[b]

                def group_msg(t, carry2):
                    sl16 = pl.ds(t * 16, 16)
                    av = plsc.bitcast(e_v[b][3, sl16], jnp.float32)
                    objs[b][sl16] = e_v[b][2, sl16]
                    for e in range(16):
                        i = t * 16 + e
                        a = av[e]
                        for j in range(8):
                            sl = pl.ds(j * 16, 16)
                            msg[i, sl] = a * (hm[i, sl] + rm[i, sl])
                    return carry2
                lax.fori_loop(0, K // 16, group_msg, 0)
                pltpu.async_copy(msg, agg.at[objs[b]], sems[b], add=True)

            pf(0, 0)
            pf(1, 1)
            wait_rows(0)
            compute_scatter(0, 0, True)
            pf(2, 0)
            wait_rows(1)
            compute_scatter(1, 1, True)

            def pair2(p, carry):
                pf(2 * p + 3, 1)
                wait_rows(0)
                compute_scatter(2 * p + 2, 0, False)
                pf(2 * p + 4, 0)
                wait_rows(1)
                compute_scatter(2 * p + 3, 1, False)
                return carry
            lax.fori_loop(0, nchunk // 2 - 1, pair2, 0)
            wait_rows(0)
            wait_scat(0)
            wait_scat(1)
            plsc.subcore_barrier()
            pltpu.sync_copy(agg.at[pl.ds(row0, rows_per_tile)],
                            pout.at[cid, pl.ds(row0, rows_per_tile)])

        zero_agg()
        plsc.subcore_barrier()
        run_pass(hm1_h, rm1_h, ph0)
        zero_agg()
        plsc.subcore_barrier()
        run_pass(hm2_h, rm2_h, ph1)

    return body


def kernel(q_sub, q_rel, hidden, edges, nodes, old_nodes_new_idx, batchsize,
           rela_embed, Ws, Wr, Wqr, b_qr, w_alpha, b_alpha, W_h):
    n_node = nodes.shape[0]
    n_edge = edges.shape[0]
    n_rel = rela_embed.shape[0]
    f32 = jnp.float32

    sub = edges[:, 4].astype(jnp.int32)
    rel = edges[:, 2].astype(jnp.int32)
    obj = edges[:, 5].astype(jnp.int32)
    ridx = edges[:, 0].astype(jnp.int32)

    npad = ((max(n_node, n_rel) + 255) // 256) * 256
    ept = ((n_edge + NW * 2 * K - 1) // (NW * 2 * K)) * 2 * K
    nchunk = ept // K
    epad = ept * NW
    pad = epad - n_edge

    def colpack(x):
        return jnp.pad(x, (0, pad)).reshape(NW, nchunk, K)
    ecols_a = jnp.stack([colpack(sub), colpack(rel), colpack(ridx)], axis=2)
    sro = [colpack(sub), colpack(rel), colpack(obj)]

    hid_p = jnp.pad(hidden.astype(f32), ((0, npad - n_node), (0, 0)))
    rel_p = jnp.pad(rela_embed.astype(f32), ((0, npad - n_rel), (0, 0)))

    nblk = npad // 256
    tbl_a, tbl_b, tbl_c = pl.pallas_call(
        _prep_body,
        grid=(nblk,),
        in_specs=[
            pl.BlockSpec((256, D), lambda i: (i, 0)),
            pl.BlockSpec((256, D), lambda i: (i, 0)),
            pl.BlockSpec((D, D), lambda i: (0, 0)),
            pl.BlockSpec((D, D), lambda i: (0, 0)),
            pl.BlockSpec((D, D), lambda i: (0, 0)),
            pl.BlockSpec((1, D), lambda i: (0, 0)),
        ],
        out_specs=[
            pl.BlockSpec((256, D), lambda i: (i, 0)),
            pl.BlockSpec((256, D), lambda i: (i, 0)),
            pl.BlockSpec((256, D), lambda i: (i, 0)),
        ],
        out_shape=[jax.ShapeDtypeStruct((npad, D), f32)] * 3,
    )(hid_p, rel_p, Ws.T.astype(f32), Wr.T.astype(f32), Wqr.T.astype(f32),
      b_qr.reshape(1, D).astype(f32))

    wal = w_alpha.reshape(-1).astype(f32)
    bal = jnp.broadcast_to(b_alpha.astype(f32), (16,))

    mesh = plsc.VectorSubcoreMesh(core_axis_name="c", subcore_axis_name="s",
                                  num_cores=NC, num_subcores=NS)
    sc_params = pltpu.CompilerParams(needs_layout_passes=False)

    i32 = jnp.int32
    alpha_fn = pl.kernel(
        _make_alpha_body(n_edge, ept),
        out_type=jax.ShapeDtypeStruct((epad,), f32),
        mesh=mesh,
        compiler_params=sc_params,
        scratch_types=[
            pltpu.VMEM((q_rel.shape[0],), i32),
            pltpu.VMEM((D,), f32),
            pltpu.VMEM((16,), f32),
            pltpu.VMEM((3, K), i32),
            pltpu.VMEM((3, K), i32),
            pltpu.VMEM((K,), i32),
            pltpu.VMEM((K,), i32),
            pltpu.VMEM((K,), i32),
            pltpu.VMEM((K,), i32),
            pltpu.VMEM((K,), i32),
            pltpu.VMEM((K,), i32),
            pltpu.VMEM((16, 16), f32),
            pltpu.VMEM((K, D), f32),
            pltpu.VMEM((K, D), f32),
            pltpu.VMEM((K, D), f32),
            pltpu.VMEM((K, D), f32),
            pltpu.VMEM((K, D), f32),
            pltpu.VMEM((K, D), f32),
            pltpu.VMEM((ept,), f32),
            pltpu.SemaphoreType.DMA,
            pltpu.SemaphoreType.DMA,
        ],
    )
    alphas = alpha_fn(ecols_a, q_rel.astype(i32), tbl_a, tbl_b, tbl_c, wal, bal)
    abits = jax.lax.bitcast_convert_type(
        alphas, i32).reshape(NW, nchunk, 1, K)
    ecols_g = jnp.concatenate(
        [jnp.stack(sro, axis=2), abits], axis=2)

    hm1 = hidden[:, :DH].astype(f32)
    hm2 = hidden[:, DH:].astype(f32)
    rm1 = rela_embed[:, :DH].astype(f32)
    rm2 = rela_embed[:, DH:].astype(f32)

    agg_fn = pl.kernel(
        _make_agg_body(ept, npad),
        out_type=(jax.ShapeDtypeStruct((NC, npad, DH), f32),
                  jax.ShapeDtypeStruct((NC, npad, DH), f32)),
        mesh=mesh,
        compiler_params=sc_params,
        scratch_types=[
            pltpu.VMEM_SHARED((npad, DH), f32),
            pltpu.VMEM((4, K), i32),
            pltpu.VMEM((4, K), i32),
            pltpu.VMEM((K,), i32),
            pltpu.VMEM((K,), i32),
            pltpu.VMEM((K,), i32),
            pltpu.VMEM((K,), i32),
            pltpu.VMEM((K,), i32),
            pltpu.VMEM((K,), i32),
            pltpu.VMEM((K, DH), f32),
            pltpu.VMEM((K, DH), f32),
            pltpu.VMEM((K, DH), f32),
            pltpu.VMEM((K, DH), f32),
            pltpu.VMEM((K, DH), f32),
            pltpu.VMEM((K, DH), f32),
            pltpu.VMEM((64, DH), f32),
            pltpu.SemaphoreType.DMA,
            pltpu.SemaphoreType.DMA,
            pltpu.SemaphoreType.DMA,
            pltpu.SemaphoreType.DMA,
        ],
    )
    ph0, ph1 = agg_fn(ecols_g, hm1, rm1, hm2, rm2)

    whT = W_h.T.astype(f32)
    out = pl.pallas_call(
        _final_body,
        grid=(nblk,),
        in_specs=[
            pl.BlockSpec((NC, 256, DH), lambda i: (0, i, 0)),
            pl.BlockSpec((NC, 256, DH), lambda i: (0, i, 0)),
            pl.BlockSpec((DH, D), lambda i: (0, 0)),
            pl.BlockSpec((DH, D), lambda i: (0, 0)),
        ],
        out_specs=pl.BlockSpec((256, D), lambda i: (i, 0)),
        out_shape=jax.ShapeDtypeStruct((npad, D), f32),
    )(ph0, ph1, whT[:DH], whT[DH:])

    return out[:n_node]
